# baseline (speedup 1.0000x reference)
import functools

import jax
import jax.numpy as jnp
from jax import lax
from jax.experimental import pallas as pl
from jax.experimental.pallas import tpu as pltpu
from jax.experimental.pallas import tpu_sc as plsc

_NC = 2
_NS = 16
_NW = _NC * _NS
_L = 16
_K = 128


def _sc_mesh():
    return plsc.VectorSubcoreMesh(core_axis_name="c", subcore_axis_name="s")


def _fill(ref, n, value16):
    def body(i, _):
        ref[pl.ds(i * _L, _L)] = value16
        return 0
    lax.fori_loop(0, n // _L, body, 0)


def _num_chunks(e_rows, wid):
    return (e_rows - 1 - wid) // _NW + 1


def _make_sc_degree(n_pad, e_rows):
    rows_per_tile = n_pad // _NS

    @functools.partial(
        pl.kernel,
        out_type=jax.ShapeDtypeStruct((_NC, n_pad), jnp.float32),
        mesh=_sc_mesh(),
        scratch_types=[
            [pltpu.VMEM((_K,), jnp.int32) for _ in range(2)],
            pltpu.VMEM((_K,), jnp.float32),
            pltpu.VMEM((rows_per_tile,), jnp.float32),
            pltpu.VMEM_SHARED((n_pad,), jnp.float32),
            [pltpu.SemaphoreType.DMA for _ in range(2)],
        ],
    )
    def k(e_hbm, out_hbm, ibufs, ones_v, zb, acc_sp, sems):
        c = lax.axis_index("c")
        s = lax.axis_index("s")
        wid = c * _NS + s
        base = s * rows_per_tile
        nch = _num_chunks(e_rows, wid)
        _fill(ones_v, _K, jnp.ones((_L,), jnp.float32))
        _fill(zb, rows_per_tile, jnp.zeros((_L,), jnp.float32))
        pltpu.sync_copy(zb, acc_sp.at[pl.ds(base, rows_per_tile)])
        plsc.subcore_barrier()

        def idx(j, v):
            off = (wid + j * _NW) * _K
            return pltpu.make_async_copy(
                e_hbm.at[1, pl.ds(off, _K)], ibufs[v], sems[v])

        idx(0, 0).start()

        def outer(i, _):
            j0 = i * 2
            for v in range(2):
                j = j0 + v

                @pl.when(j < nch)
                def _():
                    idx(j, v).wait()

                    @pl.when(j + 1 < nch)
                    def _():
                        idx(j + 1, (v + 1) % 2).start()
                    pltpu.sync_copy(ones_v, acc_sp.at[ibufs[v]], add=True)
            return 0
        lax.fori_loop(0, (nch + 1) // 2, outer, 0)

        plsc.subcore_barrier()
        pltpu.sync_copy(acc_sp.at[pl.ds(base, rows_per_tile)],
                        out_hbm.at[c, pl.ds(base, rows_per_tile)])

    return k


def _make_sc_segment_sum(d, n_pad, e_rows, n_buf):
    rows_per_tile = n_pad // _NS

    @functools.partial(
        pl.kernel,
        out_type=jax.ShapeDtypeStruct((_NC, n_pad, d), jnp.float32),
        mesh=_sc_mesh(),
        scratch_types=[
            [pltpu.VMEM((_K,), jnp.int32) for _ in range(n_buf)],
            [pltpu.VMEM((_K,), jnp.int32) for _ in range(n_buf)],
            [pltpu.VMEM((_K, d), jnp.float32) for _ in range(n_buf)],
            pltpu.VMEM_SHARED((n_pad, d), jnp.float32),
            [pltpu.SemaphoreType.DMA for _ in range(3 * n_buf)],
        ],
    )
    def k(xw_hbm, e_hbm, out_hbm, sbufs, dbufs, rows, acc_sp, sems):
        c = lax.axis_index("c")
        s = lax.axis_index("s")
        wid = c * _NS + s
        base = s * rows_per_tile
        nch = _num_chunks(e_rows, wid)
        ssems, dsems, gsems = (sems[0:n_buf], sems[n_buf:2 * n_buf],
                               sems[2 * n_buf:])

        zero16 = jnp.zeros((_L,), jnp.float32)

        def zrow(i, _):
            def zcol(j, _):
                rows[0][i, pl.ds(j * _L, _L)] = zero16
                return 0
            lax.fori_loop(0, d // _L, zcol, 0)
            return 0
        lax.fori_loop(0, _K, zrow, 0)
        done = 0
        while done < rows_per_tile:
            nrow = min(_K, rows_per_tile - done)
            pltpu.sync_copy(rows[0].at[pl.ds(0, nrow)],
                            acc_sp.at[pl.ds(base + done, nrow)])
            done += nrow
        plsc.subcore_barrier()

        def idx(j, v):
            off = (wid + j * _NW) * _K
            return (pltpu.make_async_copy(
                        e_hbm.at[0, pl.ds(off, _K)], sbufs[v], ssems[v]),
                    pltpu.make_async_copy(
                        e_hbm.at[1, pl.ds(off, _K)], dbufs[v], dsems[v]))

        def gather(v):
            return pltpu.make_async_copy(
                xw_hbm.at[sbufs[v]], rows[v], gsems[v])

        for p in range(n_buf):
            for cp in idx(p, p):
                @pl.when(p < nch)
                def _(cp=cp):
                    cp.start()
        for p in range(n_buf - 1):
            @pl.when(p < nch)
            def _(p=p):
                si, di = idx(p, p)
                si.wait()
                di.wait()
                gather(p).start()

        def outer(i, _):
            j0 = i * n_buf
            for v in range(n_buf):
                j = j0 + v

                @pl.when(j < nch)
                def _(v=v, j=j):
                    gather(v).wait()
                    nxt = j + n_buf - 1

                    @pl.when(nxt < nch)
                    def _(v=v, nxt=nxt):
                        w = (v + n_buf - 1) % n_buf
                        si2, di2 = idx(nxt, w)
                        si2.wait()
                        di2.wait()
                        gather(w).start()
                    pltpu.sync_copy(rows[v], acc_sp.at[dbufs[v]], add=True)
                    si, di = idx(j + n_buf, v)

                    @pl.when(j + n_buf < nch)
                    def _(si=si, di=di):
                        si.start()
                        di.start()
            return 0
        lax.fori_loop(0, (nch + n_buf - 1) // n_buf, outer, 0)

        plsc.subcore_barrier()
        pltpu.sync_copy(acc_sp.at[pl.ds(base, rows_per_tile)],
                        out_hbm.at[c, pl.ds(base, rows_per_tile)])

    return k


def _tc_prescale(x, w1, deg_t, rows):
    n, d_in = x.shape
    d_out = w1.shape[1]

    def body(x_b, w_b, deg_b, xwp_b, dinv_b):
        deg = deg_b[:, 0:1] + deg_b[:, 1:2] + 1.0
        dinv = lax.rsqrt(deg)
        xw = jnp.dot(x_b[...], w_b[...], preferred_element_type=jnp.float32)
        xwp_b[...] = xw * dinv
        dinv_b[...] = dinv

    return pl.pallas_call(
        body,
        grid=(n // rows,),
        in_specs=[
            pl.BlockSpec((rows, d_in), lambda i: (i, 0)),
            pl.BlockSpec((d_in, d_out), lambda i: (0, 0)),
            pl.BlockSpec((rows, _NC), lambda i: (i, 0)),
        ],
        out_specs=[
            pl.BlockSpec((rows, d_out), lambda i: (i, 0)),
            pl.BlockSpec((rows, 1), lambda i: (i, 0)),
        ],
        out_shape=[
            jax.ShapeDtypeStruct((n, d_out), jnp.float32),
            jax.ShapeDtypeStruct((n, 1), jnp.float32),
        ],
    )(x, w1, deg_t)


def _tc_mid(acc, xwp, dinv, b_in, w, rows):
    n, d = xwp.shape
    d_out = w.shape[1]

    def body(a0_b, a1_b, xwp_b, dinv_b, b_b, w_b, out_b):
        h = jnp.maximum(
            (a0_b[0] + a1_b[0] + xwp_b[...]) * dinv_b[...] + b_b[...], 0.0)
        out_b[...] = jnp.dot(h, w_b[...],
                             preferred_element_type=jnp.float32) * dinv_b[...]

    return pl.pallas_call(
        body,
        grid=(n // rows,),
        in_specs=[
            pl.BlockSpec((1, rows, d), lambda i: (0, i, 0)),
            pl.BlockSpec((1, rows, d), lambda i: (1, i, 0)),
            pl.BlockSpec((rows, d), lambda i: (i, 0)),
            pl.BlockSpec((rows, 1), lambda i: (i, 0)),
            pl.BlockSpec((1, d), lambda i: (0, 0)),
            pl.BlockSpec((d, d_out), lambda i: (0, 0)),
        ],
        out_specs=pl.BlockSpec((rows, d_out), lambda i: (i, 0)),
        out_shape=jax.ShapeDtypeStruct((n, d_out), jnp.float32),
    )(acc, acc, xwp, dinv, b_in.reshape(1, d), w)


def _tc_final(acc, xwp, dinv, b_in, w, b_out, rows):
    n, d = xwp.shape
    d_out = w.shape[1]

    def body(a0_b, a1_b, xwp_b, dinv_b, b_b, w_b, bo_b, out_b):
        h = jnp.maximum(
            (a0_b[0] + a1_b[0] + xwp_b[...]) * dinv_b[...] + b_b[...], 0.0)
        z = jnp.dot(h, w_b[...], preferred_element_type=jnp.float32) + bo_b[...]
        out_b[...] = jax.nn.sigmoid(z)

    return pl.pallas_call(
        body,
        grid=(n // rows,),
        in_specs=[
            pl.BlockSpec((1, rows, d), lambda i: (0, i, 0)),
            pl.BlockSpec((1, rows, d), lambda i: (1, i, 0)),
            pl.BlockSpec((rows, d), lambda i: (i, 0)),
            pl.BlockSpec((rows, 1), lambda i: (i, 0)),
            pl.BlockSpec((1, d), lambda i: (0, 0)),
            pl.BlockSpec((d, d_out), lambda i: (0, 0)),
            pl.BlockSpec((1, d_out), lambda i: (0, 0)),
        ],
        out_specs=pl.BlockSpec((rows, d_out), lambda i: (i, 0)),
        out_shape=jax.ShapeDtypeStruct((n, d_out), jnp.float32),
    )(acc, acc, xwp, dinv, b_in.reshape(1, d), w, b_out.reshape(1, d_out))


def kernel(x, edge_index, W1, b1, W2, b2, Wlin, blin):
    n, _ = x.shape
    e = edge_index.shape[1]
    assert e % _K == 0
    e_rows = e // _K
    n_buf = 3

    n_pad_deg = -(-n // (_NS * 128)) * (_NS * 128)
    n_pad_seg = -(-n // (_NS * 8)) * (_NS * 8)

    rows = 2000
    assert n % rows == 0

    deg_parts = _make_sc_degree(n_pad_deg, e_rows)(edge_index)
    deg_t = deg_parts.T[:n]

    seg = _make_sc_segment_sum(x.shape[1], n_pad_seg, e_rows, n_buf)
    xwp1, dinv = _tc_prescale(x, W1, deg_t, rows)
    acc1 = seg(xwp1, edge_index)
    xwp2 = _tc_mid(acc1, xwp1, dinv, b1, W2, rows)
    acc2 = seg(xwp2, edge_index)
    return _tc_final(acc2, xwp2, dinv, b2, Wlin, blin, rows)


# speedup vs baseline: 1.1001x; 1.1001x over previous
import functools

import jax
import jax.numpy as jnp
from jax import lax
from jax.experimental import pallas as pl
from jax.experimental.pallas import tpu as pltpu
from jax.experimental.pallas import tpu_sc as plsc

_NC = 2
_NS = 16
_NW = _NC * _NS
_L = 16
_KD = 128


def _sc_mesh():
    return plsc.VectorSubcoreMesh(core_axis_name="c", subcore_axis_name="s")


def _fill(ref, n, value16):
    def body(i, _):
        ref[pl.ds(i * _L, _L)] = value16
        return 0
    lax.fori_loop(0, n // _L, body, 0)


def _make_sc_degree(n_pad, e_rows):
    rows_per_tile = n_pad // _NS

    @functools.partial(
        pl.kernel,
        out_type=jax.ShapeDtypeStruct((_NC, n_pad), jnp.float32),
        mesh=_sc_mesh(),
        scratch_types=[
            [pltpu.VMEM((_KD,), jnp.int32) for _ in range(2)],
            pltpu.VMEM((_KD,), jnp.float32),
            pltpu.VMEM((rows_per_tile,), jnp.float32),
            pltpu.VMEM_SHARED((n_pad,), jnp.float32),
            [pltpu.SemaphoreType.DMA for _ in range(2)],
        ],
    )
    def k(e_hbm, out_hbm, ibufs, ones_v, zb, acc_sp, sems):
        c = lax.axis_index("c")
        s = lax.axis_index("s")
        wid = c * _NS + s
        base = s * rows_per_tile
        nch = (e_rows - 1 - wid) // _NW + 1
        _fill(ones_v, _KD, jnp.ones((_L,), jnp.float32))
        _fill(zb, rows_per_tile, jnp.zeros((_L,), jnp.float32))
        pltpu.sync_copy(zb, acc_sp.at[pl.ds(base, rows_per_tile)])
        plsc.subcore_barrier()

        def idx(j, v):
            off = (wid + j * _NW) * _KD
            return pltpu.make_async_copy(
                e_hbm.at[1, pl.ds(off, _KD)], ibufs[v], sems[v])

        idx(0, 0).start()

        def outer(i, _):
            j0 = i * 2
            for v in range(2):
                j = j0 + v

                @pl.when(j < nch)
                def _(v=v, j=j):
                    idx(j, v).wait()

                    @pl.when(j + 1 < nch)
                    def _(v=v, j=j):
                        idx(j + 1, (v + 1) % 2).start()
                    pltpu.sync_copy(ones_v, acc_sp.at[ibufs[v]], add=True)
            return 0
        lax.fori_loop(0, (nch + 1) // 2, outer, 0)

        plsc.subcore_barrier()
        pltpu.sync_copy(acc_sp.at[pl.ds(base, rows_per_tile)],
                        out_hbm.at[c, pl.ds(base, rows_per_tile)])

    return k


def _make_sc_segment_sum(d, n_pad, k_chunk, n_chunks, n_blocks, n_buf):
    rows_per_tile = n_pad // _NS
    sb = n_chunks // n_blocks
    assert sb % n_buf == 0 and sb % 8 == 0

    @functools.partial(
        pl.kernel,
        out_type=jax.ShapeDtypeStruct((_NC, n_pad, d), jnp.float32),
        mesh=_sc_mesh(),
        scratch_types=[
            [pltpu.VMEM((sb, k_chunk), jnp.int32) for _ in range(2)],
            [pltpu.VMEM((sb, k_chunk), jnp.int32) for _ in range(2)],
            [pltpu.VMEM((k_chunk, d), jnp.float32) for _ in range(n_buf)],
            pltpu.VMEM_SHARED((n_pad, d), jnp.float32),
            [pltpu.SemaphoreType.DMA for _ in range(4 + n_buf)],
        ],
    )
    def k(xw_hbm, e_hbm, out_hbm, srcbs, dstbs, rows, acc_sp, sems):
        c = lax.axis_index("c")
        s = lax.axis_index("s")
        wid = c * _NS + s
        base = s * rows_per_tile
        ssems, dsems, gsems = sems[0:2], sems[2:4], sems[4:]

        zero16 = jnp.zeros((_L,), jnp.float32)

        def zrow(i, _):
            def zcol(j, _):
                rows[0][i, pl.ds(j * _L, _L)] = zero16
                return 0
            lax.fori_loop(0, d // _L, zcol, 0)
            return 0
        lax.fori_loop(0, k_chunk, zrow, 0)
        zc = (k_chunk // 8) * 8
        done = 0
        while done < rows_per_tile:
            nrow = min(zc, rows_per_tile - done)
            pltpu.sync_copy(rows[0].at[pl.ds(0, nrow)],
                            acc_sp.at[pl.ds(base + done, nrow)])
            done += nrow
        plsc.subcore_barrier()

        def idx_block(sd, b, buf, sem):
            return pltpu.make_async_copy(
                e_hbm.at[sd, wid, pl.ds(b * sb, sb)], buf, sem)

        def gather(srcb, jj, v):
            return pltpu.make_async_copy(
                xw_hbm.at[srcb.at[jj]], rows[v], gsems[v])

        idx_block(0, 0, srcbs[0], ssems[0]).start()
        idx_block(1, 0, dstbs[0], dsems[0]).start()
        for b in range(n_blocks):
            srcb, dstb = srcbs[b % 2], dstbs[b % 2]
            idx_block(0, b, srcb, ssems[b % 2]).wait()
            idx_block(1, b, dstb, dsems[b % 2]).wait()
            if b + 1 < n_blocks:
                idx_block(0, b + 1, srcbs[(b + 1) % 2],
                          ssems[(b + 1) % 2]).start()
                idx_block(1, b + 1, dstbs[(b + 1) % 2],
                          dsems[(b + 1) % 2]).start()

            for v in range(n_buf - 1):
                gather(srcb, v, v).start()

            def outer(i, _):
                j0 = i * n_buf
                for v in range(n_buf):
                    j = j0 + v
                    gather(srcb, j, v).wait()
                    nxt = j + n_buf - 1

                    @pl.when(nxt < sb)
                    def _(v=v, nxt=nxt):
                        gather(srcb, nxt, (v - 1) % n_buf).start()
                    pltpu.sync_copy(rows[v], acc_sp.at[dstb.at[j]], add=True)
                return 0
            lax.fori_loop(0, sb // n_buf, outer, 0)

        plsc.subcore_barrier()
        pltpu.sync_copy(acc_sp.at[pl.ds(base, rows_per_tile)],
                        out_hbm.at[c, pl.ds(base, rows_per_tile)])

    return k


def _tc_prescale(x, w1, deg_t, rows):
    n, d_in = x.shape
    d_out = w1.shape[1]

    def body(x_b, w_b, deg_b, xwp_b, dinv_b):
        deg = deg_b[:, 0:1] + deg_b[:, 1:2] + 1.0
        dinv = lax.rsqrt(deg)
        xw = jnp.dot(x_b[...], w_b[...], preferred_element_type=jnp.float32)
        xwp_b[...] = xw * dinv
        dinv_b[...] = dinv

    return pl.pallas_call(
        body,
        grid=(n // rows,),
        in_specs=[
            pl.BlockSpec((rows, d_in), lambda i: (i, 0)),
            pl.BlockSpec((d_in, d_out), lambda i: (0, 0)),
            pl.BlockSpec((rows, _NC), lambda i: (i, 0)),
        ],
        out_specs=[
            pl.BlockSpec((rows, d_out), lambda i: (i, 0)),
            pl.BlockSpec((rows, 1), lambda i: (i, 0)),
        ],
        out_shape=[
            jax.ShapeDtypeStruct((n, d_out), jnp.float32),
            jax.ShapeDtypeStruct((n, 1), jnp.float32),
        ],
    )(x, w1, deg_t)


def _tc_mid(acc, xwp, dinv, b_in, w, rows):
    n, d = xwp.shape
    d_out = w.shape[1]

    def body(a0_b, a1_b, xwp_b, dinv_b, b_b, w_b, out_b):
        h = jnp.maximum(
            (a0_b[0] + a1_b[0] + xwp_b[...]) * dinv_b[...] + b_b[...], 0.0)
        out_b[...] = jnp.dot(h, w_b[...],
                             preferred_element_type=jnp.float32) * dinv_b[...]

    return pl.pallas_call(
        body,
        grid=(n // rows,),
        in_specs=[
            pl.BlockSpec((1, rows, d), lambda i: (0, i, 0)),
            pl.BlockSpec((1, rows, d), lambda i: (1, i, 0)),
            pl.BlockSpec((rows, d), lambda i: (i, 0)),
            pl.BlockSpec((rows, 1), lambda i: (i, 0)),
            pl.BlockSpec((1, d), lambda i: (0, 0)),
            pl.BlockSpec((d, d_out), lambda i: (0, 0)),
        ],
        out_specs=pl.BlockSpec((rows, d_out), lambda i: (i, 0)),
        out_shape=jax.ShapeDtypeStruct((n, d_out), jnp.float32),
    )(acc, acc, xwp, dinv, b_in.reshape(1, d), w)


def _tc_final(acc, xwp, dinv, b_in, w, b_out, rows):
    n, d = xwp.shape
    d_out = w.shape[1]

    def body(a0_b, a1_b, xwp_b, dinv_b, b_b, w_b, bo_b, out_b):
        h = jnp.maximum(
            (a0_b[0] + a1_b[0] + xwp_b[...]) * dinv_b[...] + b_b[...], 0.0)
        z = jnp.dot(h, w_b[...], preferred_element_type=jnp.float32) + bo_b[...]
        out_b[...] = jax.nn.sigmoid(z)

    return pl.pallas_call(
        body,
        grid=(n // rows,),
        in_specs=[
            pl.BlockSpec((1, rows, d), lambda i: (0, i, 0)),
            pl.BlockSpec((1, rows, d), lambda i: (1, i, 0)),
            pl.BlockSpec((rows, d), lambda i: (i, 0)),
            pl.BlockSpec((rows, 1), lambda i: (i, 0)),
            pl.BlockSpec((1, d), lambda i: (0, 0)),
            pl.BlockSpec((d, d_out), lambda i: (0, 0)),
            pl.BlockSpec((1, d_out), lambda i: (0, 0)),
        ],
        out_specs=pl.BlockSpec((rows, d_out), lambda i: (i, 0)),
        out_shape=jax.ShapeDtypeStruct((n, d_out), jnp.float32),
    )(acc, acc, xwp, dinv, b_in.reshape(1, d), w, b_out.reshape(1, d_out))


def kernel(x, edge_index, W1, b1, W2, b2, Wlin, blin):
    n, _ = x.shape
    e = edge_index.shape[1]
    assert e % _KD == 0

    per_tile = e // _NW
    k_chunk = 50
    assert per_tile % k_chunk == 0
    n_chunks = per_tile // k_chunk
    n_blocks = 5
    n_buf = 4

    n_pad_deg = -(-n // (_NS * 128)) * (_NS * 128)
    n_pad_seg = -(-n // (_NS * 8)) * (_NS * 8)

    rows = 2000
    assert n % rows == 0

    e4 = edge_index.reshape(2, _NW, n_chunks, k_chunk)

    deg_parts = _make_sc_degree(n_pad_deg, e // _KD)(edge_index)
    deg_t = deg_parts.T[:n]

    seg = _make_sc_segment_sum(x.shape[1], n_pad_seg, k_chunk,
                               n_chunks, n_blocks, n_buf)
    xwp1, dinv = _tc_prescale(x, W1, deg_t, rows)
    acc1 = seg(xwp1, e4)
    xwp2 = _tc_mid(acc1, xwp1, dinv, b1, W2, rows)
    acc2 = seg(xwp2, e4)
    return _tc_final(acc2, xwp2, dinv, b2, Wlin, blin, rows)


# speedup vs baseline: 1.1355x; 1.0322x over previous
import functools

import jax
import jax.numpy as jnp
from jax import lax
from jax.experimental import pallas as pl
from jax.experimental.pallas import tpu as pltpu
from jax.experimental.pallas import tpu_sc as plsc

_NC = 2
_NS = 16
_NW = _NC * _NS
_L = 16
_KD = 128


def _sc_mesh():
    return plsc.VectorSubcoreMesh(core_axis_name="c", subcore_axis_name="s")


def _fill(ref, n, value16):
    def body(i, _):
        ref[pl.ds(i * _L, _L)] = value16
        return 0
    lax.fori_loop(0, n // _L, body, 0)


def _make_sc_degree(n_pad, k_chunk, n_chunks):
    rows_per_tile = n_pad // _NS
    ones_n = -(-k_chunk // _L) * _L

    @functools.partial(
        pl.kernel,
        out_type=jax.ShapeDtypeStruct((_NC, n_pad), jnp.float32),
        mesh=_sc_mesh(),
        scratch_types=[
            pltpu.VMEM((n_chunks, k_chunk), jnp.int32),
            pltpu.VMEM((ones_n,), jnp.float32),
            pltpu.VMEM((rows_per_tile,), jnp.float32),
            pltpu.VMEM_SHARED((n_pad,), jnp.float32),
        ],
    )
    def k(e_hbm, out_hbm, dst_v, ones_v, zb, acc_sp):
        c = lax.axis_index("c")
        s = lax.axis_index("s")
        wid = c * _NS + s
        base = s * rows_per_tile
        _fill(ones_v, ones_n, jnp.ones((_L,), jnp.float32))
        _fill(zb, rows_per_tile, jnp.zeros((_L,), jnp.float32))
        pltpu.sync_copy(zb, acc_sp.at[pl.ds(base, rows_per_tile)])
        plsc.subcore_barrier()
        pltpu.sync_copy(e_hbm.at[1, wid], dst_v)

        def body(j, _):
            pltpu.sync_copy(ones_v.at[pl.ds(0, k_chunk)],
                            acc_sp.at[dst_v.at[j]], add=True)
            return 0
        lax.fori_loop(0, n_chunks, body, 0)

        plsc.subcore_barrier()
        pltpu.sync_copy(acc_sp.at[pl.ds(base, rows_per_tile)],
                        out_hbm.at[c, pl.ds(base, rows_per_tile)])

    return k


def _make_sc_segment_sum(d, n_pad, k_chunk, n_chunks, n_blocks, n_buf):
    rows_per_tile = n_pad // _NS
    sb = n_chunks // n_blocks
    assert sb % n_buf == 0 and sb % 8 == 0

    @functools.partial(
        pl.kernel,
        out_type=jax.ShapeDtypeStruct((_NC, n_pad, d), jnp.float32),
        mesh=_sc_mesh(),
        scratch_types=[
            [pltpu.VMEM((sb, k_chunk), jnp.int32) for _ in range(2)],
            [pltpu.VMEM((sb, k_chunk), jnp.int32) for _ in range(2)],
            [pltpu.VMEM((k_chunk, d), jnp.float32) for _ in range(n_buf)],
            pltpu.VMEM_SHARED((n_pad, d), jnp.float32),
            [pltpu.SemaphoreType.DMA for _ in range(4 + n_buf)],
        ],
    )
    def k(xw_hbm, e_hbm, out_hbm, srcbs, dstbs, rows, acc_sp, sems):
        c = lax.axis_index("c")
        s = lax.axis_index("s")
        wid = c * _NS + s
        base = s * rows_per_tile
        ssems, dsems, gsems = sems[0:2], sems[2:4], sems[4:]

        zero16 = jnp.zeros((_L,), jnp.float32)

        def zrow(i, _):
            def zcol(j, _):
                rows[0][i, pl.ds(j * _L, _L)] = zero16
                return 0
            lax.fori_loop(0, d // _L, zcol, 0)
            return 0
        lax.fori_loop(0, k_chunk, zrow, 0)
        zc = (k_chunk // 8) * 8
        done = 0
        while done < rows_per_tile:
            nrow = min(zc, rows_per_tile - done)
            pltpu.sync_copy(rows[0].at[pl.ds(0, nrow)],
                            acc_sp.at[pl.ds(base + done, nrow)])
            done += nrow
        plsc.subcore_barrier()

        def idx_block(sd, b, buf, sem):
            return pltpu.make_async_copy(
                e_hbm.at[sd, wid, pl.ds(b * sb, sb)], buf, sem)

        def gather(srcb, jj, v):
            return pltpu.make_async_copy(
                xw_hbm.at[srcb.at[jj]], rows[v], gsems[v])

        idx_block(0, 0, srcbs[0], ssems[0]).start()
        idx_block(1, 0, dstbs[0], dsems[0]).start()
        for b in range(n_blocks):
            srcb, dstb = srcbs[b % 2], dstbs[b % 2]
            idx_block(0, b, srcb, ssems[b % 2]).wait()
            idx_block(1, b, dstb, dsems[b % 2]).wait()
            if b + 1 < n_blocks:
                idx_block(0, b + 1, srcbs[(b + 1) % 2],
                          ssems[(b + 1) % 2]).start()
                idx_block(1, b + 1, dstbs[(b + 1) % 2],
                          dsems[(b + 1) % 2]).start()

            for v in range(n_buf - 1):
                gather(srcb, v, v).start()

            def outer(i, _):
                j0 = i * n_buf
                for v in range(n_buf):
                    j = j0 + v
                    gather(srcb, j, v).wait()
                    nxt = j + n_buf - 1

                    @pl.when(nxt < sb)
                    def _(v=v, nxt=nxt):
                        gather(srcb, nxt, (v - 1) % n_buf).start()
                    pltpu.sync_copy(rows[v], acc_sp.at[dstb.at[j]], add=True)
                return 0
            lax.fori_loop(0, sb // n_buf, outer, 0)

        plsc.subcore_barrier()
        pltpu.sync_copy(acc_sp.at[pl.ds(base, rows_per_tile)],
                        out_hbm.at[c, pl.ds(base, rows_per_tile)])

    return k


def _tc_prescale(x, w1, deg_t, rows):
    n, d_in = x.shape
    d_out = w1.shape[1]

    def body(x_b, w_b, deg_b, xwp_b, dinv_b):
        deg = deg_b[:, 0:1] + deg_b[:, 1:2] + 1.0
        dinv = lax.rsqrt(deg)
        xw = jnp.dot(x_b[...], w_b[...], preferred_element_type=jnp.float32)
        xwp_b[...] = xw * dinv
        dinv_b[...] = dinv

    return pl.pallas_call(
        body,
        grid=(n // rows,),
        in_specs=[
            pl.BlockSpec((rows, d_in), lambda i: (i, 0)),
            pl.BlockSpec((d_in, d_out), lambda i: (0, 0)),
            pl.BlockSpec((rows, _NC), lambda i: (i, 0)),
        ],
        out_specs=[
            pl.BlockSpec((rows, d_out), lambda i: (i, 0)),
            pl.BlockSpec((rows, 1), lambda i: (i, 0)),
        ],
        out_shape=[
            jax.ShapeDtypeStruct((n, d_out), jnp.float32),
            jax.ShapeDtypeStruct((n, 1), jnp.float32),
        ],
    )(x, w1, deg_t)


def _tc_mid(acc, xwp, dinv, b_in, w, rows):
    n, d = xwp.shape
    d_out = w.shape[1]

    def body(a0_b, a1_b, xwp_b, dinv_b, b_b, w_b, out_b):
        h = jnp.maximum(
            (a0_b[0] + a1_b[0] + xwp_b[...]) * dinv_b[...] + b_b[...], 0.0)
        out_b[...] = jnp.dot(h, w_b[...],
                             preferred_element_type=jnp.float32) * dinv_b[...]

    return pl.pallas_call(
        body,
        grid=(n // rows,),
        in_specs=[
            pl.BlockSpec((1, rows, d), lambda i: (0, i, 0)),
            pl.BlockSpec((1, rows, d), lambda i: (1, i, 0)),
            pl.BlockSpec((rows, d), lambda i: (i, 0)),
            pl.BlockSpec((rows, 1), lambda i: (i, 0)),
            pl.BlockSpec((1, d), lambda i: (0, 0)),
            pl.BlockSpec((d, d_out), lambda i: (0, 0)),
        ],
        out_specs=pl.BlockSpec((rows, d_out), lambda i: (i, 0)),
        out_shape=jax.ShapeDtypeStruct((n, d_out), jnp.float32),
    )(acc, acc, xwp, dinv, b_in.reshape(1, d), w)


def _tc_final(acc, xwp, dinv, b_in, w, b_out, rows):
    n, d = xwp.shape
    d_out = w.shape[1]

    def body(a0_b, a1_b, xwp_b, dinv_b, b_b, w_b, bo_b, out_b):
        h = jnp.maximum(
            (a0_b[0] + a1_b[0] + xwp_b[...]) * dinv_b[...] + b_b[...], 0.0)
        z = jnp.dot(h, w_b[...], preferred_element_type=jnp.float32) + bo_b[...]
        out_b[...] = jax.nn.sigmoid(z)

    return pl.pallas_call(
        body,
        grid=(n // rows,),
        in_specs=[
            pl.BlockSpec((1, rows, d), lambda i: (0, i, 0)),
            pl.BlockSpec((1, rows, d), lambda i: (1, i, 0)),
            pl.BlockSpec((rows, d), lambda i: (i, 0)),
            pl.BlockSpec((rows, 1), lambda i: (i, 0)),
            pl.BlockSpec((1, d), lambda i: (0, 0)),
            pl.BlockSpec((d, d_out), lambda i: (0, 0)),
            pl.BlockSpec((1, d_out), lambda i: (0, 0)),
        ],
        out_specs=pl.BlockSpec((rows, d_out), lambda i: (i, 0)),
        out_shape=jax.ShapeDtypeStruct((n, d_out), jnp.float32),
    )(acc, acc, xwp, dinv, b_in.reshape(1, d), w, b_out.reshape(1, d_out))


def kernel(x, edge_index, W1, b1, W2, b2, Wlin, blin):
    n, _ = x.shape
    e = edge_index.shape[1]
    assert e % _KD == 0

    per_tile = e // _NW
    k_chunk = 50
    assert per_tile % k_chunk == 0
    n_chunks = per_tile // k_chunk
    n_blocks = 5
    n_buf = 4

    n_pad_deg = -(-n // (_NS * 128)) * (_NS * 128)
    n_pad_seg = -(-n // (_NS * 8)) * (_NS * 8)

    rows = 2000
    assert n % rows == 0

    e4 = edge_index.reshape(2, _NW, n_chunks, k_chunk)

    deg_parts = _make_sc_degree(n_pad_deg, k_chunk, n_chunks)(e4)
    deg_t = deg_parts.T[:n]

    seg = _make_sc_segment_sum(x.shape[1], n_pad_seg, k_chunk,
                               n_chunks, n_blocks, n_buf)
    xwp1, dinv = _tc_prescale(x, W1, deg_t, rows)
    acc1 = seg(xwp1, e4)
    xwp2 = _tc_mid(acc1, xwp1, dinv, b1, W2, rows)
    acc2 = seg(xwp2, e4)
    return _tc_final(acc2, xwp2, dinv, b2, Wlin, blin, rows)


# speedup vs baseline: 1.1359x; 1.0003x over previous
import functools

import jax
import jax.numpy as jnp
from jax import lax
from jax.experimental import pallas as pl
from jax.experimental.pallas import tpu as pltpu
from jax.experimental.pallas import tpu_sc as plsc

_NC = 2
_NS = 16
_NW = _NC * _NS
_L = 16


def _sc_mesh():
    return plsc.VectorSubcoreMesh(core_axis_name="c", subcore_axis_name="s")


def _fill(ref, n, value16):
    def body(i, _):
        ref[pl.ds(i * _L, _L)] = value16
        return 0
    lax.fori_loop(0, n // _L, body, 0)


def _make_sc_degree(n_pad, k_chunk, n_chunks):
    rows_per_tile = n_pad // _NS
    ones_n = -(-k_chunk // _L) * _L

    @functools.partial(
        pl.kernel,
        out_type=jax.ShapeDtypeStruct((_NC, n_pad), jnp.float32),
        mesh=_sc_mesh(),
        scratch_types=[
            pltpu.VMEM((n_chunks, k_chunk), jnp.int32),
            pltpu.VMEM((ones_n,), jnp.float32),
            pltpu.VMEM((rows_per_tile,), jnp.float32),
            pltpu.VMEM_SHARED((n_pad,), jnp.float32),
        ],
    )
    def k(e_hbm, out_hbm, dst_v, ones_v, zb, acc_sp):
        c = lax.axis_index("c")
        s = lax.axis_index("s")
        wid = c * _NS + s
        base = s * rows_per_tile
        _fill(ones_v, ones_n, jnp.ones((_L,), jnp.float32))
        _fill(zb, rows_per_tile, jnp.zeros((_L,), jnp.float32))
        pltpu.sync_copy(zb, acc_sp.at[pl.ds(base, rows_per_tile)])
        plsc.subcore_barrier()
        pltpu.sync_copy(e_hbm.at[1, wid], dst_v)

        def body(j, _):
            pltpu.sync_copy(ones_v.at[pl.ds(0, k_chunk)],
                            acc_sp.at[dst_v.at[j]], add=True)
            return 0
        lax.fori_loop(0, n_chunks, body, 0)

        plsc.subcore_barrier()
        pltpu.sync_copy(acc_sp.at[pl.ds(base, rows_per_tile)],
                        out_hbm.at[c, pl.ds(base, rows_per_tile)])

    return k


def _make_sc_segment_sum(d, n_pad, k_chunk, n_chunks, n_blocks, n_buf):
    rows_per_tile = n_pad // _NS
    sb = n_chunks // n_blocks
    assert sb % n_buf == 0 and sb % 8 == 0

    @functools.partial(
        pl.kernel,
        out_type=jax.ShapeDtypeStruct((_NC, n_pad, d), jnp.float32),
        mesh=_sc_mesh(),
        scratch_types=[
            [pltpu.VMEM((sb, k_chunk), jnp.int32) for _ in range(2)],
            [pltpu.VMEM((sb, k_chunk), jnp.int32) for _ in range(2)],
            [pltpu.VMEM((k_chunk, d), jnp.float32) for _ in range(n_buf)],
            pltpu.VMEM_SHARED((n_pad, d), jnp.float32),
            [pltpu.SemaphoreType.DMA for _ in range(4 + n_buf)],
        ],
    )
    def k(xw_hbm, e_hbm, out_hbm, srcbs, dstbs, rows, acc_sp, sems):
        c = lax.axis_index("c")
        s = lax.axis_index("s")
        wid = c * _NS + s
        base = s * rows_per_tile
        ssems, dsems, gsems = sems[0:2], sems[2:4], sems[4:]

        zero16 = jnp.zeros((_L,), jnp.float32)

        def zrow(i, _):
            def zcol(j, _):
                rows[0][i, pl.ds(j * _L, _L)] = zero16
                return 0
            lax.fori_loop(0, d // _L, zcol, 0)
            return 0
        lax.fori_loop(0, k_chunk, zrow, 0)
        zc = (k_chunk // 8) * 8
        done = 0
        while done < rows_per_tile:
            nrow = min(zc, rows_per_tile - done)
            pltpu.sync_copy(rows[0].at[pl.ds(0, nrow)],
                            acc_sp.at[pl.ds(base + done, nrow)])
            done += nrow
        plsc.subcore_barrier()

        def idx_block(sd, b, buf, sem):
            return pltpu.make_async_copy(
                e_hbm.at[sd, wid, pl.ds(b * sb, sb)], buf, sem)

        def gather(srcb, jj, v):
            return pltpu.make_async_copy(
                xw_hbm.at[srcb.at[jj]], rows[v], gsems[v])

        idx_block(0, 0, srcbs[0], ssems[0]).start()
        idx_block(1, 0, dstbs[0], dsems[0]).start()
        for b in range(n_blocks):
            srcb, dstb = srcbs[b % 2], dstbs[b % 2]
            idx_block(0, b, srcb, ssems[b % 2]).wait()
            idx_block(1, b, dstb, dsems[b % 2]).wait()
            if b + 1 < n_blocks:
                idx_block(0, b + 1, srcbs[(b + 1) % 2],
                          ssems[(b + 1) % 2]).start()
                idx_block(1, b + 1, dstbs[(b + 1) % 2],
                          dsems[(b + 1) % 2]).start()

            for v in range(n_buf - 1):
                gather(srcb, v, v).start()

            def outer(i, _):
                j0 = i * n_buf
                for v in range(n_buf):
                    j = j0 + v
                    gather(srcb, j, v).wait()
                    nxt = j + n_buf - 1

                    @pl.when(nxt < sb)
                    def _(v=v, nxt=nxt):
                        gather(srcb, nxt, (v - 1) % n_buf).start()
                    pltpu.sync_copy(rows[v], acc_sp.at[dstb.at[j]], add=True)
                return 0
            lax.fori_loop(0, sb // n_buf, outer, 0)

        plsc.subcore_barrier()
        pltpu.sync_copy(acc_sp.at[pl.ds(base, rows_per_tile)],
                        out_hbm.at[c, pl.ds(base, rows_per_tile)])

    return k


def _tc_prescale(x, w1, deg_t, rows):
    n, d_in = x.shape
    d_out = w1.shape[1]

    def body(x_b, w_b, deg_b, xwp_b, dinv_b):
        deg = deg_b[:, 0:1] + deg_b[:, 1:2] + 1.0
        dinv = lax.rsqrt(deg)
        xw = jnp.dot(x_b[...], w_b[...], preferred_element_type=jnp.float32)
        xwp_b[...] = xw * dinv
        dinv_b[...] = dinv

    return pl.pallas_call(
        body,
        grid=(n // rows,),
        in_specs=[
            pl.BlockSpec((rows, d_in), lambda i: (i, 0)),
            pl.BlockSpec((d_in, d_out), lambda i: (0, 0)),
            pl.BlockSpec((rows, _NC), lambda i: (i, 0)),
        ],
        out_specs=[
            pl.BlockSpec((rows, d_out), lambda i: (i, 0)),
            pl.BlockSpec((rows, 1), lambda i: (i, 0)),
        ],
        out_shape=[
            jax.ShapeDtypeStruct((n, d_out), jnp.float32),
            jax.ShapeDtypeStruct((n, 1), jnp.float32),
        ],
    )(x, w1, deg_t)


def _tc_mid(acc, xwp, dinv, b_in, w, rows):
    n, d = xwp.shape
    d_out = w.shape[1]

    def body(a0_b, a1_b, xwp_b, dinv_b, b_b, w_b, out_b):
        h = jnp.maximum(
            (a0_b[0] + a1_b[0] + xwp_b[...]) * dinv_b[...] + b_b[...], 0.0)
        out_b[...] = jnp.dot(h, w_b[...],
                             preferred_element_type=jnp.float32) * dinv_b[...]

    return pl.pallas_call(
        body,
        grid=(n // rows,),
        in_specs=[
            pl.BlockSpec((1, rows, d), lambda i: (0, i, 0)),
            pl.BlockSpec((1, rows, d), lambda i: (1, i, 0)),
            pl.BlockSpec((rows, d), lambda i: (i, 0)),
            pl.BlockSpec((rows, 1), lambda i: (i, 0)),
            pl.BlockSpec((1, d), lambda i: (0, 0)),
            pl.BlockSpec((d, d_out), lambda i: (0, 0)),
        ],
        out_specs=pl.BlockSpec((rows, d_out), lambda i: (i, 0)),
        out_shape=jax.ShapeDtypeStruct((n, d_out), jnp.float32),
    )(acc, acc, xwp, dinv, b_in.reshape(1, d), w)


def _tc_final(acc, xwp, dinv, b_in, w, b_out, rows):
    n, d = xwp.shape
    d_out = w.shape[1]

    def body(a0_b, a1_b, xwp_b, dinv_b, b_b, w_b, bo_b, out_b):
        h = jnp.maximum(
            (a0_b[0] + a1_b[0] + xwp_b[...]) * dinv_b[...] + b_b[...], 0.0)
        z = jnp.dot(h, w_b[...], preferred_element_type=jnp.float32) + bo_b[...]
        out_b[...] = jax.nn.sigmoid(z)

    return pl.pallas_call(
        body,
        grid=(n // rows,),
        in_specs=[
            pl.BlockSpec((1, rows, d), lambda i: (0, i, 0)),
            pl.BlockSpec((1, rows, d), lambda i: (1, i, 0)),
            pl.BlockSpec((rows, d), lambda i: (i, 0)),
            pl.BlockSpec((rows, 1), lambda i: (i, 0)),
            pl.BlockSpec((1, d), lambda i: (0, 0)),
            pl.BlockSpec((d, d_out), lambda i: (0, 0)),
            pl.BlockSpec((1, d_out), lambda i: (0, 0)),
        ],
        out_specs=pl.BlockSpec((rows, d_out), lambda i: (i, 0)),
        out_shape=jax.ShapeDtypeStruct((n, d_out), jnp.float32),
    )(acc, acc, xwp, dinv, b_in.reshape(1, d), w, b_out.reshape(1, d_out))


def kernel(x, edge_index, W1, b1, W2, b2, Wlin, blin):
    n, _ = x.shape
    e = edge_index.shape[1]
    assert e % _NW == 0

    per_tile = e // _NW
    k_chunk = 50
    assert per_tile % k_chunk == 0
    n_chunks = per_tile // k_chunk
    n_blocks = 5
    n_buf = 4

    n_pad_deg = -(-n // (_NS * 128)) * (_NS * 128)
    n_pad_seg = -(-n // (_NS * 8)) * (_NS * 8)

    rows = 2000
    assert n % rows == 0

    e4 = edge_index.reshape(2, _NW, n_chunks, k_chunk)

    deg_parts = _make_sc_degree(n_pad_deg, k_chunk, n_chunks)(e4)
    deg_t = deg_parts.T[:n]

    seg = _make_sc_segment_sum(x.shape[1], n_pad_seg, k_chunk,
                               n_chunks, n_blocks, n_buf)
    xwp1, dinv = _tc_prescale(x, W1, deg_t, rows)
    acc1 = seg(xwp1, e4)
    xwp2 = _tc_mid(acc1, xwp1, dinv, b1, W2, rows)
    acc2 = seg(xwp2, e4)
    return _tc_final(acc2, xwp2, dinv, b2, Wlin, blin, rows)
